# baseline (speedup 1.0000x reference)
import functools
import math

import jax
import jax.numpy as jnp
import numpy as np
from jax import lax
from jax.experimental import pallas as pl
from jax.experimental.pallas import tpu as pltpu
from jax.experimental.pallas import tpu_sc as plsc

_VOCAB = 100000
_D = 768
_B = 4
_S = 4096
_NC = 2
_NS = 16
_NW = _NC * _NS
_POS_PER_W = _S // _NW
_RCHUNK = 32
_NCHUNK = _POS_PER_W // _RCHUNK
_LANES = 16
_NVEC = _D // _LANES
_SCALE = math.sqrt(_D)


def _make_pe_np(max_len, d_model):
    pe = np.zeros((max_len, d_model), dtype=np.float32)
    position = np.arange(0, max_len, dtype=np.float32)[:, None]
    div_term = np.exp(
        np.arange(0, d_model, 2, dtype=np.float32) * -(math.log(10000.0) / d_model)
    )
    pe[:, 0::2] = np.sin(position * div_term)
    pe[:, 1::2] = np.cos(position * div_term)
    return pe


_PE = _make_pe_np(_S, _D)


def _body(x_hbm, table_hbm, pe_hbm, out_hbm, idx_v, pe_v, rows_v, sem):
    wid = lax.axis_index("s") * _NC + lax.axis_index("c")
    pos0 = wid * _POS_PER_W

    for b in range(_B):
        pltpu.sync_copy(x_hbm.at[pl.ds(b * _S + pos0, _POS_PER_W)], idx_v.at[b])

    for c in range(_NCHUNK):
        pltpu.sync_copy(pe_hbm.at[pl.ds(pos0 + c * _RCHUNK, _RCHUNK)], pe_v)
        for b in range(_B):
            pltpu.async_copy(
                table_hbm.at[idx_v.at[b, pl.ds(c * _RCHUNK, _RCHUNK)]],
                rows_v,
                sem,
            ).wait()

            def row_body(i, _):
                for j in range(_NVEC):
                    sl = pl.ds(j * _LANES, _LANES)
                    rows_v[i, sl] = rows_v[i, sl] * _SCALE + pe_v[i, sl]
                return 0

            lax.fori_loop(0, _RCHUNK, row_body, 0)
            pltpu.sync_copy(
                rows_v, out_hbm.at[pl.ds(b * _S + pos0 + c * _RCHUNK, _RCHUNK)]
            )


def _build(interpret=False):
    mesh = plsc.VectorSubcoreMesh(core_axis_name="c", subcore_axis_name="s")
    return pl.kernel(
        _body,
        out_type=jax.ShapeDtypeStruct((_B * _S, _D), jnp.float32),
        mesh=mesh,
        scratch_types=[
            pltpu.VMEM((_B, _POS_PER_W), jnp.int32),
            pltpu.VMEM((_RCHUNK, _D), jnp.float32),
            pltpu.VMEM((_RCHUNK, _D), jnp.float32),
            pltpu.SemaphoreType.DMA,
        ],
        interpret=interpret,
    )


_sc_embed = _build()


def kernel(x, table):
    x_flat = x.reshape(_B * _S).astype(jnp.int32)
    out = _sc_embed(x_flat, table, jnp.asarray(_PE))
    return out.reshape(_B, _S, _D)


# speedup vs baseline: 1.3683x; 1.3683x over previous
import functools
import math

import jax
import jax.numpy as jnp
import numpy as np
from jax import lax
from jax.experimental import pallas as pl
from jax.experimental.pallas import tpu as pltpu
from jax.experimental.pallas import tpu_sc as plsc

_VOCAB = 100000
_D = 768
_B = 4
_S = 4096
_NC = 2
_NS = 16
_NW = _NC * _NS
_POS_PER_W = _S // _NW
_RCHUNK = 32
_NCHUNK = _POS_PER_W // _RCHUNK
_NUNITS = _NCHUNK * _B
_LANES = 16
_NVEC = _D // _LANES
_SCALE = math.sqrt(_D)


def _make_pe_np(max_len, d_model):
    pe = np.zeros((max_len, d_model), dtype=np.float32)
    position = np.arange(0, max_len, dtype=np.float32)[:, None]
    div_term = np.exp(
        np.arange(0, d_model, 2, dtype=np.float32) * -(math.log(10000.0) / d_model)
    )
    pe[:, 0::2] = np.sin(position * div_term)
    pe[:, 1::2] = np.cos(position * div_term)
    return pe


_PE = _make_pe_np(_S, _D)


def _body(x_hbm, table_hbm, pe_hbm, out_hbm, idx_v, pe_v, gbuf, obuf, gsem, ssem):
    wid = lax.axis_index("s") * _NC + lax.axis_index("c")
    pos0 = wid * _POS_PER_W

    for b in range(_B):
        pltpu.sync_copy(x_hbm.at[pl.ds(b * _S + pos0, _POS_PER_W)], idx_v.at[b])

    def gather(u):
        c, b = divmod(u, _B)
        return pltpu.async_copy(
            table_hbm.at[idx_v.at[b, pl.ds(c * _RCHUNK, _RCHUNK)]],
            gbuf.at[u % 2],
            gsem,
        )

    def scatter(u):
        c, b = divmod(u, _B)
        return pltpu.async_copy(
            obuf.at[u % 2],
            out_hbm.at[pl.ds(b * _S + pos0 + c * _RCHUNK, _RCHUNK)],
            ssem,
        )

    g = {0: gather(0), 1: gather(1)}
    s = {}
    for u in range(_NUNITS):
        c, b = divmod(u, _B)
        if b == 0:
            pltpu.sync_copy(pe_hbm.at[pl.ds(pos0 + c * _RCHUNK, _RCHUNK)], pe_v)
        g[u].wait()
        if u >= 2:
            s[u - 2].wait()

        def row_body(i, _, _u=u):
            for j in range(_NVEC):
                sl = pl.ds(j * _LANES, _LANES)
                obuf[_u % 2, i, sl] = gbuf[_u % 2, i, sl] * _SCALE + pe_v[i, sl]
            return 0

        lax.fori_loop(0, _RCHUNK, row_body, 0)
        s[u] = scatter(u)
        if u + 2 < _NUNITS:
            g[u + 2] = gather(u + 2)
    s[_NUNITS - 2].wait()
    s[_NUNITS - 1].wait()


def _build(interpret=False):
    mesh = plsc.VectorSubcoreMesh(core_axis_name="c", subcore_axis_name="s")
    return pl.kernel(
        _body,
        out_type=jax.ShapeDtypeStruct((_B * _S, _D), jnp.float32),
        mesh=mesh,
        scratch_types=[
            pltpu.VMEM((_B, _POS_PER_W), jnp.int32),
            pltpu.VMEM((_RCHUNK, _D), jnp.float32),
            pltpu.VMEM((2, _RCHUNK, _D), jnp.float32),
            pltpu.VMEM((2, _RCHUNK, _D), jnp.float32),
            pltpu.SemaphoreType.DMA,
            pltpu.SemaphoreType.DMA,
        ],
        interpret=interpret,
    )


_sc_embed = _build()


def kernel(x, table):
    x_flat = x.reshape(_B * _S).astype(jnp.int32)
    out = _sc_embed(x_flat, table, jnp.asarray(_PE))
    return out.reshape(_B, _S, _D)
